# baseline (speedup 1.0000x reference)
import functools

import jax
import jax.numpy as jnp
from jax import lax
from jax.experimental import pallas as pl
from jax.experimental.pallas import tpu as pltpu
from jax.experimental.pallas import tpu_sc as plsc

H = 128
NV = 10000
NE = 5000
NNZ = 320000
NC = 2
NS = 16
NW = NC * NS
PAIRS_PER_W = NNZ // NW

CH1 = 96
NCHUNK1 = -(-PAIRS_PER_W // CH1)
PADW = NCHUNK1 * CH1
PADTOT = float(NW * (PADW - PAIRS_PER_W))

CH2 = 80
NCHUNK2 = PAIRS_PER_W // CH2

NE_PAD = NS * 320
NV_PAD = NS * 640


def _mesh():
    return plsc.VectorSubcoreMesh(core_axis_name="c", subcore_axis_name="s",
                                  num_cores=NC, num_subcores=NS)


def _fill(ref, rows, cols, val):
    vec = jnp.full((16,), val, jnp.float32)

    def body(i, carry):
        for j in range(cols // 16):
            ref[i, pl.ds(j * 16, 16)] = vec
        return carry

    lax.fori_loop(0, rows, body, 0)


def _sc1_body(x_hbm, vert_hbm, edge_hbm,
              a_out, ce_out, cv_out,
              vidx_all, eidx_all, rows0, rows1, ones_v, cstage_v,
              a_sh, ce_sh, cv_sh, gsa, gsb):
    cid = lax.axis_index("c")
    sid = lax.axis_index("s")
    wid = sid * NC + cid

    _fill(rows0, CH1, H, 0.0)
    _fill(cstage_v, 640, 16, 0.0)
    _fill(ones_v, CH1, 16, 1.0)
    stripes = [(o, min(CH1, 320 - o)) for o in range(0, 320, CH1)]
    for off, n in stripes:
        pltpu.sync_copy(rows0.at[pl.ds(0, n)],
                        a_sh.at[pl.ds(sid * 320 + off, n)])
    pltpu.sync_copy(cstage_v.at[pl.ds(0, 320)], ce_sh.at[pl.ds(sid * 320, 320)])
    pltpu.sync_copy(cstage_v, cv_sh.at[pl.ds(sid * 640, 640)])
    pltpu.sync_copy(vert_hbm.at[wid], vidx_all)
    pltpu.sync_copy(edge_hbm.at[wid], eidx_all)
    plsc.subcore_barrier()

    pltpu.async_copy(x_hbm.at[vidx_all.at[0]], rows0, gsa)

    def scatter(i, rows):
        pltpu.sync_copy(rows, a_sh.at[eidx_all.at[i]], add=True)
        pltpu.sync_copy(ones_v, ce_sh.at[eidx_all.at[i]], add=True)
        pltpu.sync_copy(ones_v, cv_sh.at[vidx_all.at[i]], add=True)

    def step(k, carry):
        i0 = 2 * k
        pltpu.async_copy(x_hbm.at[vidx_all.at[i0 + 1]], rows1, gsb)
        pltpu.make_async_copy(x_hbm.at[vidx_all.at[i0]], rows0, gsa).wait()
        scatter(i0, rows0)
        pltpu.async_copy(x_hbm.at[vidx_all.at[i0 + 2]], rows0, gsa)
        pltpu.make_async_copy(x_hbm.at[vidx_all.at[i0]], rows1, gsb).wait()
        scatter(i0 + 1, rows1)
        return carry

    lax.fori_loop(0, (NCHUNK1 - 1) // 2, step, 0)
    pltpu.make_async_copy(x_hbm.at[vidx_all.at[0]], rows0, gsa).wait()
    scatter(NCHUNK1 - 1, rows0)
    plsc.subcore_barrier()

    for off, n in stripes:
        pltpu.sync_copy(a_sh.at[pl.ds(sid * 320 + off, n)],
                        rows0.at[pl.ds(0, n)])
        pltpu.sync_copy(rows0.at[pl.ds(0, n)],
                        a_out.at[pl.ds(cid * NE_PAD + sid * 320 + off, n)])
    pltpu.sync_copy(ce_sh.at[pl.ds(sid * 320, 320)], cstage_v.at[pl.ds(0, 320)])
    pltpu.sync_copy(cstage_v.at[pl.ds(0, 320)],
                    ce_out.at[pl.ds(cid * NE_PAD + sid * 320, 320)])
    pltpu.sync_copy(cv_sh.at[pl.ds(sid * 640, 640)], cstage_v)
    pltpu.sync_copy(cstage_v,
                    cv_out.at[pl.ds(cid * NV_PAD + sid * 640, 640)])


@functools.cache
def _sc1():
  return pl.kernel(
    _sc1_body,
    out_type=(
        jax.ShapeDtypeStruct((NC * NE_PAD, H), jnp.float32),
        jax.ShapeDtypeStruct((NC * NE_PAD, 16), jnp.float32),
        jax.ShapeDtypeStruct((NC * NV_PAD, 16), jnp.float32),
    ),
    mesh=_mesh(),
    compiler_params=pltpu.CompilerParams(use_tc_tiling_on_sc=False),
    scratch_types=[
        pltpu.VMEM((NCHUNK1, CH1), jnp.int32),
        pltpu.VMEM((NCHUNK1, CH1), jnp.int32),
        pltpu.VMEM((CH1, H), jnp.float32),
        pltpu.VMEM((CH1, H), jnp.float32),
        pltpu.VMEM((CH1, 16), jnp.float32),
        pltpu.VMEM((640, 16), jnp.float32),
        pltpu.VMEM_SHARED((NE_PAD, H), jnp.float32),
        pltpu.VMEM_SHARED((NE_PAD, 16), jnp.float32),
        pltpu.VMEM_SHARED((NV_PAD, 16), jnp.float32),
        pltpu.SemaphoreType.DMA,
        pltpu.SemaphoreType.DMA,
    ],
  )


def _sc2_body(e2_hbm, vert_hbm, edge_hbm,
              g_out,
              vidx_all, eidx_all, rows0, rows1,
              g_sh, gsa, gsb):
    cid = lax.axis_index("c")
    sid = lax.axis_index("s")
    wid = sid * NC + cid

    _fill(rows0, CH2, H, 0.0)
    for j in range(8):
        pltpu.sync_copy(rows0, g_sh.at[pl.ds(sid * 640 + j * CH2, CH2)])
    pltpu.sync_copy(vert_hbm.at[wid], vidx_all)
    pltpu.sync_copy(edge_hbm.at[wid], eidx_all)
    plsc.subcore_barrier()

    pltpu.async_copy(e2_hbm.at[eidx_all.at[0]], rows0, gsa)

    def step(k, carry):
        i0 = 2 * k
        pltpu.async_copy(e2_hbm.at[eidx_all.at[i0 + 1]], rows1, gsb)
        pltpu.make_async_copy(e2_hbm.at[eidx_all.at[i0]], rows0, gsa).wait()
        pltpu.sync_copy(rows0, g_sh.at[vidx_all.at[i0]], add=True)
        pltpu.async_copy(e2_hbm.at[eidx_all.at[i0 + 2]], rows0, gsa)
        pltpu.make_async_copy(e2_hbm.at[eidx_all.at[i0]], rows1, gsb).wait()
        pltpu.sync_copy(rows1, g_sh.at[vidx_all.at[i0 + 1]], add=True)
        return carry

    lax.fori_loop(0, (NCHUNK2 - 1) // 2, step, 0)
    pltpu.make_async_copy(e2_hbm.at[eidx_all.at[0]], rows0, gsa).wait()
    pltpu.sync_copy(rows0, g_sh.at[vidx_all.at[NCHUNK2 - 1]], add=True)
    plsc.subcore_barrier()

    for j in range(8):
        pltpu.sync_copy(g_sh.at[pl.ds(sid * 640 + j * CH2, CH2)], rows0)
        pltpu.sync_copy(rows0,
                        g_out.at[pl.ds(cid * NV_PAD + sid * 640 + j * CH2,
                                       CH2)])


@functools.cache
def _sc2():
  return pl.kernel(
    _sc2_body,
    out_type=jax.ShapeDtypeStruct((NC * NV_PAD, H), jnp.float32),
    mesh=_mesh(),
    compiler_params=pltpu.CompilerParams(use_tc_tiling_on_sc=False),
    scratch_types=[
        pltpu.VMEM((NCHUNK2, CH2), jnp.int32),
        pltpu.VMEM((NCHUNK2, CH2), jnp.int32),
        pltpu.VMEM((CH2, H), jnp.float32),
        pltpu.VMEM((CH2, H), jnp.float32),
        pltpu.VMEM_SHARED((NV_PAD, H), jnp.float32),
        pltpu.SemaphoreType.DMA,
        pltpu.SemaphoreType.DMA,
    ],
  )


def _dot(a, b):
    return jnp.dot(a, b, preferred_element_type=jnp.float32)


def _tc1_body(a_ref, ce_ref, e_ref, w1_ref, b1_ref, w2_ref, b2_ref, o_ref):
    a = a_ref[0] + a_ref[1]
    cnt = ce_ref[0, :, 0:1] + ce_ref[1, :, 0:1]
    ind = (cnt > 0.0).astype(jnp.float32)
    am = a / jnp.maximum(cnt, 1.0)
    e = e_ref[...]
    me = _dot(am, w1_ref[0:H]) + ind * (_dot(e, w1_ref[H:2 * H]) + b1_ref[...])
    o_ref[...] = _dot(e, w2_ref[0:H]) + _dot(me, w2_ref[H:2 * H]) + b2_ref[...]


def _tc2_body(x_ref, g_ref, cv_ref, w3_ref, b3_ref, w4_ref, b4_ref, o_ref):
    g = g_ref[0] + g_ref[1]
    cnt = cv_ref[0, :, 0:1] + cv_ref[1, :, 0:1]
    row = lax.broadcasted_iota(jnp.int32, cnt.shape, 0)
    first = (pl.program_id(0) == 0) & (row == 0)
    cnt = cnt - jnp.where(first, PADTOT, 0.0)
    ind = (cnt > 0.0).astype(jnp.float32)
    gm = g / jnp.maximum(cnt, 1.0)
    x = x_ref[...]
    mv = ind * (_dot(x, w3_ref[0:H]) + b3_ref[...]) + _dot(gm, w3_ref[H:2 * H])
    o_ref[...] = _dot(x, w4_ref[0:H]) + _dot(mv, w4_ref[H:2 * H]) + b4_ref[...]


def _full(shape):
    return pl.BlockSpec(shape, lambda i: (0,) * len(shape))


def _rows(R):
    return pl.BlockSpec((R, H), lambda i: (i, 0))


def _make_tc1():
    R = 1000
    return pl.pallas_call(
        _tc1_body,
        grid=(NE // R,),
        in_specs=[
            pl.BlockSpec((NC, R, H), lambda i: (0, i, 0)),
            pl.BlockSpec((NC, R, 16), lambda i: (0, i, 0)),
            _rows(R),
            _full((2 * H, H)),
            _full((1, H)),
            _full((2 * H, H)),
            _full((1, H)),
        ],
        out_specs=_rows(R),
        out_shape=jax.ShapeDtypeStruct((NE, H), jnp.float32),
    )


def _make_tc2():
    R = 1000
    return pl.pallas_call(
        _tc2_body,
        grid=(NV // R,),
        in_specs=[
            _rows(R),
            pl.BlockSpec((NC, R, H), lambda i: (0, i, 0)),
            pl.BlockSpec((NC, R, 16), lambda i: (0, i, 0)),
            _full((2 * H, H)),
            _full((1, H)),
            _full((2 * H, H)),
            _full((1, H)),
        ],
        out_specs=_rows(R),
        out_shape=jax.ShapeDtypeStruct((NV, H), jnp.float32),
    )


@jax.jit
def _run(X, E, vertex, edges, W1, b1, W2, b2, W3, b3, W4, b4):
    vertex = vertex.astype(jnp.int32).reshape(NW, PAIRS_PER_W)
    edges = edges.astype(jnp.int32).reshape(NW, PAIRS_PER_W)
    v_g = jnp.pad(vertex, ((0, 0), (0, PADW - PAIRS_PER_W))
                  ).reshape(NW, NCHUNK1, CH1)
    e_s = jnp.pad(edges, ((0, 0), (0, PADW - PAIRS_PER_W)),
                  constant_values=NE_PAD - 1).reshape(NW, NCHUNK1, CH1)
    v_2 = vertex.reshape(NW, NCHUNK2, CH2)
    e_2 = edges.reshape(NW, NCHUNK2, CH2)

    a_p, ce_p, cv_p = _sc1()(X, v_g, e_s)
    a_p = a_p.reshape(NC, NE_PAD, H)
    ce_p = ce_p.reshape(NC, NE_PAD, 16)
    cv_p = cv_p.reshape(NC, NV_PAD, 16)
    e2 = _make_tc1()(a_p, ce_p, E,
                     W1, b1.reshape(1, H), W2, b2.reshape(1, H))
    g_p = _sc2()(e2, v_2, e_2)
    g_p = g_p.reshape(NC, NV_PAD, H)
    x2 = _make_tc2()(X, g_p, cv_p,
                     W3, b3.reshape(1, H), W4, b4.reshape(1, H))
    return x2, e2


def kernel(X, E, vertex, edges, W1, b1, W2, b2, W3, b3, W4, b4):
    return _run(X, E, vertex, edges, W1, b1, W2, b2, W3, b3, W4, b4)


# speedup vs baseline: 1.3523x; 1.3523x over previous
import functools

import jax
import jax.numpy as jnp
from jax import lax
from jax.experimental import pallas as pl
from jax.experimental.pallas import tpu as pltpu
from jax.experimental.pallas import tpu_sc as plsc

H = 128
NV = 10000
NE = 5000
NNZ = 320000
NC = 2
NS = 16
NW = NC * NS
PAIRS_PER_W = NNZ // NW

CH1 = 80
NCHUNK1 = -(-PAIRS_PER_W // CH1)
PADW = NCHUNK1 * CH1
PADTOT = float(NW * (PADW - PAIRS_PER_W))

CH2 = 80
NCHUNK2 = PAIRS_PER_W // CH2

NE_PAD = NS * 320
NV_PAD = NS * 640


def _mesh():
    return plsc.VectorSubcoreMesh(core_axis_name="c", subcore_axis_name="s",
                                  num_cores=NC, num_subcores=NS)


def _fill(ref, rows, cols, val):
    vec = jnp.full((16,), val, jnp.float32)

    def body(i, carry):
        for j in range(cols // 16):
            ref[i, pl.ds(j * 16, 16)] = vec
        return carry

    lax.fori_loop(0, rows, body, 0)


def _sc1_body(x_hbm, vert_hbm, edge_hbm,
              a_out, ce_out, cv_out,
              vidx_all, eidx_all, rows0, rows1, ones_v, cstage_v,
              a_sh, ce_sh, cv_sh, gsa, gsb, ssem):
    cid = lax.axis_index("c")
    sid = lax.axis_index("s")
    wid = sid * NC + cid

    idxv = pltpu.async_copy(vert_hbm.at[wid], vidx_all, gsa)
    idxe = pltpu.async_copy(edge_hbm.at[wid], eidx_all, gsb)
    _fill(rows0, CH1, H, 0.0)
    _fill(cstage_v, 640, 16, 0.0)
    _fill(ones_v, CH1, 16, 1.0)
    stripes = [(o, min(CH1, 320 - o)) for o in range(0, 320, CH1)]
    for off, n in stripes:
        pltpu.sync_copy(rows0.at[pl.ds(0, n)],
                        a_sh.at[pl.ds(sid * 320 + off, n)])
    pltpu.sync_copy(cstage_v.at[pl.ds(0, 320)], ce_sh.at[pl.ds(sid * 320, 320)])
    pltpu.sync_copy(cstage_v, cv_sh.at[pl.ds(sid * 640, 640)])
    idxv.wait()
    idxe.wait()
    plsc.subcore_barrier()

    pltpu.async_copy(x_hbm.at[vidx_all.at[0]], rows0, gsa)

    def scatter(i, rows):
        pltpu.async_copy(rows, a_sh.at[eidx_all.at[i]], ssem, add=True)
        pltpu.async_copy(ones_v, ce_sh.at[eidx_all.at[i]], ssem, add=True)
        pltpu.async_copy(ones_v, cv_sh.at[vidx_all.at[i]], ssem, add=True)
        pltpu.make_async_copy(rows, a_sh.at[eidx_all.at[i]], ssem).wait()
        pltpu.make_async_copy(ones_v, ce_sh.at[eidx_all.at[i]], ssem).wait()
        pltpu.make_async_copy(ones_v, cv_sh.at[vidx_all.at[i]], ssem).wait()

    def step(k, carry):
        i0 = 2 * k
        pltpu.async_copy(x_hbm.at[vidx_all.at[i0 + 1]], rows1, gsb)
        pltpu.make_async_copy(x_hbm.at[vidx_all.at[i0]], rows0, gsa).wait()
        scatter(i0, rows0)
        pltpu.async_copy(x_hbm.at[vidx_all.at[i0 + 2]], rows0, gsa)
        pltpu.make_async_copy(x_hbm.at[vidx_all.at[i0]], rows1, gsb).wait()
        scatter(i0 + 1, rows1)
        return carry

    lax.fori_loop(0, (NCHUNK1 - 1) // 2, step, 0)
    pltpu.make_async_copy(x_hbm.at[vidx_all.at[0]], rows0, gsa).wait()
    scatter(NCHUNK1 - 1, rows0)
    plsc.subcore_barrier()

    for off, n in stripes:
        pltpu.sync_copy(a_sh.at[pl.ds(sid * 320 + off, n)],
                        rows0.at[pl.ds(0, n)])
        pltpu.sync_copy(rows0.at[pl.ds(0, n)],
                        a_out.at[pl.ds(cid * NE_PAD + sid * 320 + off, n)])
    pltpu.sync_copy(ce_sh.at[pl.ds(sid * 320, 320)], cstage_v.at[pl.ds(0, 320)])
    pltpu.sync_copy(cstage_v.at[pl.ds(0, 320)],
                    ce_out.at[pl.ds(cid * NE_PAD + sid * 320, 320)])
    pltpu.sync_copy(cv_sh.at[pl.ds(sid * 640, 640)], cstage_v)
    pltpu.sync_copy(cstage_v,
                    cv_out.at[pl.ds(cid * NV_PAD + sid * 640, 640)])


@functools.cache
def _sc1():
  return pl.kernel(
    _sc1_body,
    out_type=(
        jax.ShapeDtypeStruct((NC * NE_PAD, H), jnp.float32),
        jax.ShapeDtypeStruct((NC * NE_PAD, 16), jnp.float32),
        jax.ShapeDtypeStruct((NC * NV_PAD, 16), jnp.float32),
    ),
    mesh=_mesh(),
    compiler_params=pltpu.CompilerParams(use_tc_tiling_on_sc=False),
    scratch_types=[
        pltpu.VMEM((NCHUNK1, CH1), jnp.int32),
        pltpu.VMEM((NCHUNK1, CH1), jnp.int32),
        pltpu.VMEM((CH1, H), jnp.float32),
        pltpu.VMEM((CH1, H), jnp.float32),
        pltpu.VMEM((CH1, 16), jnp.float32),
        pltpu.VMEM((640, 16), jnp.float32),
        pltpu.VMEM_SHARED((NE_PAD, H), jnp.float32),
        pltpu.VMEM_SHARED((NE_PAD, 16), jnp.float32),
        pltpu.VMEM_SHARED((NV_PAD, 16), jnp.float32),
        pltpu.SemaphoreType.DMA,
        pltpu.SemaphoreType.DMA,
        pltpu.SemaphoreType.DMA,
    ],
  )


def _sc2_body(e2_hbm, vert_hbm, edge_hbm,
              g_out,
              vidx_all, eidx_all, rows0, rows1,
              g_sh, gsa, gsb):
    cid = lax.axis_index("c")
    sid = lax.axis_index("s")
    wid = sid * NC + cid

    _fill(rows0, CH2, H, 0.0)
    for j in range(8):
        pltpu.sync_copy(rows0, g_sh.at[pl.ds(sid * 640 + j * CH2, CH2)])
    pltpu.sync_copy(vert_hbm.at[wid], vidx_all)
    pltpu.sync_copy(edge_hbm.at[wid], eidx_all)
    plsc.subcore_barrier()

    pltpu.async_copy(e2_hbm.at[eidx_all.at[0]], rows0, gsa)

    def step(k, carry):
        i0 = 2 * k
        pltpu.async_copy(e2_hbm.at[eidx_all.at[i0 + 1]], rows1, gsb)
        pltpu.make_async_copy(e2_hbm.at[eidx_all.at[i0]], rows0, gsa).wait()
        pltpu.sync_copy(rows0, g_sh.at[vidx_all.at[i0]], add=True)
        pltpu.async_copy(e2_hbm.at[eidx_all.at[i0 + 2]], rows0, gsa)
        pltpu.make_async_copy(e2_hbm.at[eidx_all.at[i0]], rows1, gsb).wait()
        pltpu.sync_copy(rows1, g_sh.at[vidx_all.at[i0 + 1]], add=True)
        return carry

    lax.fori_loop(0, (NCHUNK2 - 1) // 2, step, 0)
    pltpu.make_async_copy(e2_hbm.at[eidx_all.at[0]], rows0, gsa).wait()
    pltpu.sync_copy(rows0, g_sh.at[vidx_all.at[NCHUNK2 - 1]], add=True)
    plsc.subcore_barrier()

    for j in range(8):
        pltpu.sync_copy(g_sh.at[pl.ds(sid * 640 + j * CH2, CH2)], rows0)
        pltpu.sync_copy(rows0,
                        g_out.at[pl.ds(cid * NV_PAD + sid * 640 + j * CH2,
                                       CH2)])


@functools.cache
def _sc2():
  return pl.kernel(
    _sc2_body,
    out_type=jax.ShapeDtypeStruct((NC * NV_PAD, H), jnp.float32),
    mesh=_mesh(),
    compiler_params=pltpu.CompilerParams(use_tc_tiling_on_sc=False),
    scratch_types=[
        pltpu.VMEM((NCHUNK2, CH2), jnp.int32),
        pltpu.VMEM((NCHUNK2, CH2), jnp.int32),
        pltpu.VMEM((CH2, H), jnp.float32),
        pltpu.VMEM((CH2, H), jnp.float32),
        pltpu.VMEM_SHARED((NV_PAD, H), jnp.float32),
        pltpu.SemaphoreType.DMA,
        pltpu.SemaphoreType.DMA,
    ],
  )


def _dot(a, b):
    return jnp.dot(a, b, preferred_element_type=jnp.float32)


def _tc1_body(a_ref, ce_ref, e_ref, w1_ref, b1_ref, w2_ref, b2_ref, o_ref):
    a = a_ref[0] + a_ref[1]
    cnt = ce_ref[0, :, 0:1] + ce_ref[1, :, 0:1]
    ind = (cnt > 0.0).astype(jnp.float32)
    am = a / jnp.maximum(cnt, 1.0)
    e = e_ref[...]
    me = _dot(am, w1_ref[0:H]) + ind * (_dot(e, w1_ref[H:2 * H]) + b1_ref[...])
    o_ref[...] = _dot(e, w2_ref[0:H]) + _dot(me, w2_ref[H:2 * H]) + b2_ref[...]


def _tc2_body(x_ref, g_ref, cv_ref, w3_ref, b3_ref, w4_ref, b4_ref, o_ref):
    g = g_ref[0] + g_ref[1]
    cnt = cv_ref[0, :, 0:1] + cv_ref[1, :, 0:1]
    row = lax.broadcasted_iota(jnp.int32, cnt.shape, 0)
    first = (pl.program_id(0) == 0) & (row == 0)
    cnt = cnt - jnp.where(first, PADTOT, 0.0)
    ind = (cnt > 0.0).astype(jnp.float32)
    gm = g / jnp.maximum(cnt, 1.0)
    x = x_ref[...]
    mv = ind * (_dot(x, w3_ref[0:H]) + b3_ref[...]) + _dot(gm, w3_ref[H:2 * H])
    o_ref[...] = _dot(x, w4_ref[0:H]) + _dot(mv, w4_ref[H:2 * H]) + b4_ref[...]


def _full(shape):
    return pl.BlockSpec(shape, lambda i: (0,) * len(shape))


def _rows(R):
    return pl.BlockSpec((R, H), lambda i: (i, 0))


def _make_tc1():
    R = 1000
    return pl.pallas_call(
        _tc1_body,
        grid=(NE // R,),
        in_specs=[
            pl.BlockSpec((NC, R, H), lambda i: (0, i, 0)),
            pl.BlockSpec((NC, R, 16), lambda i: (0, i, 0)),
            _rows(R),
            _full((2 * H, H)),
            _full((1, H)),
            _full((2 * H, H)),
            _full((1, H)),
        ],
        out_specs=_rows(R),
        out_shape=jax.ShapeDtypeStruct((NE, H), jnp.float32),
    )


def _make_tc2():
    R = 1000
    return pl.pallas_call(
        _tc2_body,
        grid=(NV // R,),
        in_specs=[
            _rows(R),
            pl.BlockSpec((NC, R, H), lambda i: (0, i, 0)),
            pl.BlockSpec((NC, R, 16), lambda i: (0, i, 0)),
            _full((2 * H, H)),
            _full((1, H)),
            _full((2 * H, H)),
            _full((1, H)),
        ],
        out_specs=_rows(R),
        out_shape=jax.ShapeDtypeStruct((NV, H), jnp.float32),
    )


@jax.jit
def _run(X, E, vertex, edges, W1, b1, W2, b2, W3, b3, W4, b4):
    vertex = vertex.astype(jnp.int32).reshape(NW, PAIRS_PER_W)
    edges = edges.astype(jnp.int32).reshape(NW, PAIRS_PER_W)
    v_g = jnp.pad(vertex, ((0, 0), (0, PADW - PAIRS_PER_W))
                  ).reshape(NW, NCHUNK1, CH1)
    e_s = jnp.pad(edges, ((0, 0), (0, PADW - PAIRS_PER_W)),
                  constant_values=NE_PAD - 1).reshape(NW, NCHUNK1, CH1)
    v_2 = vertex.reshape(NW, NCHUNK2, CH2)
    e_2 = edges.reshape(NW, NCHUNK2, CH2)

    a_p, ce_p, cv_p = _sc1()(X, v_g, e_s)
    a_p = a_p.reshape(NC, NE_PAD, H)
    ce_p = ce_p.reshape(NC, NE_PAD, 16)
    cv_p = cv_p.reshape(NC, NV_PAD, 16)
    e2 = _make_tc1()(a_p, ce_p, E,
                     W1, b1.reshape(1, H), W2, b2.reshape(1, H))
    g_p = _sc2()(e2, v_2, e_2)
    g_p = g_p.reshape(NC, NV_PAD, H)
    x2 = _make_tc2()(X, g_p, cv_p,
                     W3, b3.reshape(1, H), W4, b4.reshape(1, H))
    return x2, e2


def kernel(X, E, vertex, edges, W1, b1, W2, b2, W3, b3, W4, b4):
    return _run(X, E, vertex, edges, W1, b1, W2, b2, W3, b3, W4, b4)


# speedup vs baseline: 1.3778x; 1.0189x over previous
import functools

import jax
import jax.numpy as jnp
from jax import lax
from jax.experimental import pallas as pl
from jax.experimental.pallas import tpu as pltpu
from jax.experimental.pallas import tpu_sc as plsc

H = 128
NV = 10000
NE = 5000
NNZ = 320000
NC = 2
NS = 16
NW = NC * NS
PAIRS_PER_W = NNZ // NW

CH1 = 80
NCHUNK1 = -(-PAIRS_PER_W // CH1)
PADW = NCHUNK1 * CH1
PADTOT = float(NW * (PADW - PAIRS_PER_W))

CH2 = 80
NCHUNK2 = PAIRS_PER_W // CH2

NE_PAD = NS * 320
NV_PAD = NS * 640


def _mesh():
    return plsc.VectorSubcoreMesh(core_axis_name="c", subcore_axis_name="s",
                                  num_cores=NC, num_subcores=NS)


def _fill(ref, rows, cols, val):
    vec = jnp.full((16,), val, jnp.float32)

    def body(i, carry):
        for j in range(cols // 16):
            ref[i, pl.ds(j * 16, 16)] = vec
        return carry

    lax.fori_loop(0, rows, body, 0)


def _sc1_body(x_hbm, vert_hbm, edge_hbm,
              a_out, ce_out, cv_out,
              vidx_all, eidx_all, rows0, rows1, ones_v, cstage_v,
              a_sh, ce_sh, cv_sh, gsa, gsb, ssem):
    cid = lax.axis_index("c")
    sid = lax.axis_index("s")
    wid = sid * NC + cid

    idxv = pltpu.async_copy(vert_hbm.at[wid], vidx_all, gsa)
    idxe = pltpu.async_copy(edge_hbm.at[wid], eidx_all, gsb)
    _fill(rows0, CH1, H, 0.0)
    _fill(cstage_v, 640, 16, 0.0)
    _fill(ones_v, CH1, 16, 1.0)
    stripes = [(o, min(CH1, 320 - o)) for o in range(0, 320, CH1)]
    for off, n in stripes:
        pltpu.sync_copy(rows0.at[pl.ds(0, n)],
                        a_sh.at[pl.ds(sid * 320 + off, n)])
    pltpu.sync_copy(cstage_v.at[pl.ds(0, 320)], ce_sh.at[pl.ds(sid * 320, 320)])
    pltpu.sync_copy(cstage_v, cv_sh.at[pl.ds(sid * 640, 640)])
    idxv.wait()
    idxe.wait()
    plsc.subcore_barrier()

    pltpu.async_copy(x_hbm.at[vidx_all.at[0]], rows0, gsa)

    def scatter(i, rows):
        pltpu.async_copy(rows, a_sh.at[eidx_all.at[i]], ssem, add=True)
        pltpu.async_copy(ones_v, ce_sh.at[eidx_all.at[i]], ssem, add=True)
        pltpu.async_copy(ones_v, cv_sh.at[vidx_all.at[i]], ssem, add=True)
        pltpu.make_async_copy(rows, a_sh.at[eidx_all.at[i]], ssem).wait()
        pltpu.make_async_copy(ones_v, ce_sh.at[eidx_all.at[i]], ssem).wait()
        pltpu.make_async_copy(ones_v, cv_sh.at[vidx_all.at[i]], ssem).wait()

    def step(k, carry):
        i0 = 2 * k
        pltpu.async_copy(x_hbm.at[vidx_all.at[i0 + 1]], rows1, gsb)
        pltpu.make_async_copy(x_hbm.at[vidx_all.at[i0]], rows0, gsa).wait()
        scatter(i0, rows0)
        pltpu.async_copy(x_hbm.at[vidx_all.at[i0 + 2]], rows0, gsa)
        pltpu.make_async_copy(x_hbm.at[vidx_all.at[i0]], rows1, gsb).wait()
        scatter(i0 + 1, rows1)
        return carry

    lax.fori_loop(0, (NCHUNK1 - 1) // 2, step, 0)
    pltpu.make_async_copy(x_hbm.at[vidx_all.at[0]], rows0, gsa).wait()
    scatter(NCHUNK1 - 1, rows0)
    plsc.subcore_barrier()

    for off, n in stripes:
        pltpu.sync_copy(a_sh.at[pl.ds(sid * 320 + off, n)],
                        rows0.at[pl.ds(0, n)])
        pltpu.sync_copy(rows0.at[pl.ds(0, n)],
                        a_out.at[pl.ds(cid * NE_PAD + sid * 320 + off, n)])
    pltpu.sync_copy(ce_sh.at[pl.ds(sid * 320, 320)], cstage_v.at[pl.ds(0, 320)])
    pltpu.sync_copy(cstage_v.at[pl.ds(0, 320)],
                    ce_out.at[pl.ds(cid * NE_PAD + sid * 320, 320)])
    pltpu.sync_copy(cv_sh.at[pl.ds(sid * 640, 640)], cstage_v)
    pltpu.sync_copy(cstage_v,
                    cv_out.at[pl.ds(cid * NV_PAD + sid * 640, 640)])


@functools.cache
def _sc1():
  return pl.kernel(
    _sc1_body,
    out_type=(
        jax.ShapeDtypeStruct((NC * NE_PAD, H), jnp.float32),
        jax.ShapeDtypeStruct((NC * NE_PAD, 16), jnp.float32),
        jax.ShapeDtypeStruct((NC * NV_PAD, 16), jnp.float32),
    ),
    mesh=_mesh(),
    compiler_params=pltpu.CompilerParams(use_tc_tiling_on_sc=False),
    scratch_types=[
        pltpu.VMEM((NCHUNK1, CH1), jnp.int32),
        pltpu.VMEM((NCHUNK1, CH1), jnp.int32),
        pltpu.VMEM((CH1, H), jnp.float32),
        pltpu.VMEM((CH1, H), jnp.float32),
        pltpu.VMEM((CH1, 16), jnp.float32),
        pltpu.VMEM((640, 16), jnp.float32),
        pltpu.VMEM_SHARED((NE_PAD, H), jnp.float32),
        pltpu.VMEM_SHARED((NE_PAD, 16), jnp.float32),
        pltpu.VMEM_SHARED((NV_PAD, 16), jnp.float32),
        pltpu.SemaphoreType.DMA,
        pltpu.SemaphoreType.DMA,
        pltpu.SemaphoreType.DMA,
    ],
  )


def _sc2_body(e2_hbm, vert_hbm, edge_hbm,
              g_out,
              vidx_all, eidx_all, rows0, rows1,
              g_sh, gsa, gsb):
    cid = lax.axis_index("c")
    sid = lax.axis_index("s")
    wid = sid * NC + cid

    idxv = pltpu.async_copy(vert_hbm.at[wid], vidx_all, gsa)
    idxe = pltpu.async_copy(edge_hbm.at[wid], eidx_all, gsb)
    _fill(rows0, CH2, H, 0.0)
    for j in range(8):
        pltpu.sync_copy(rows0, g_sh.at[pl.ds(sid * 640 + j * CH2, CH2)])
    idxv.wait()
    idxe.wait()
    plsc.subcore_barrier()

    pltpu.async_copy(e2_hbm.at[eidx_all.at[0]], rows0, gsa)

    def step(k, carry):
        i0 = 2 * k
        pltpu.async_copy(e2_hbm.at[eidx_all.at[i0 + 1]], rows1, gsb)
        pltpu.make_async_copy(e2_hbm.at[eidx_all.at[i0]], rows0, gsa).wait()
        pltpu.sync_copy(rows0, g_sh.at[vidx_all.at[i0]], add=True)
        pltpu.async_copy(e2_hbm.at[eidx_all.at[i0 + 2]], rows0, gsa)
        pltpu.make_async_copy(e2_hbm.at[eidx_all.at[i0]], rows1, gsb).wait()
        pltpu.sync_copy(rows1, g_sh.at[vidx_all.at[i0 + 1]], add=True)
        return carry

    lax.fori_loop(0, (NCHUNK2 - 1) // 2, step, 0)
    pltpu.make_async_copy(e2_hbm.at[eidx_all.at[0]], rows0, gsa).wait()
    pltpu.sync_copy(rows0, g_sh.at[vidx_all.at[NCHUNK2 - 1]], add=True)
    plsc.subcore_barrier()

    for j in range(8):
        pltpu.sync_copy(g_sh.at[pl.ds(sid * 640 + j * CH2, CH2)], rows0)
        pltpu.sync_copy(rows0,
                        g_out.at[pl.ds(cid * NV_PAD + sid * 640 + j * CH2,
                                       CH2)])


@functools.cache
def _sc2():
  return pl.kernel(
    _sc2_body,
    out_type=jax.ShapeDtypeStruct((NC * NV_PAD, H), jnp.float32),
    mesh=_mesh(),
    compiler_params=pltpu.CompilerParams(use_tc_tiling_on_sc=False),
    scratch_types=[
        pltpu.VMEM((NCHUNK2, CH2), jnp.int32),
        pltpu.VMEM((NCHUNK2, CH2), jnp.int32),
        pltpu.VMEM((CH2, H), jnp.float32),
        pltpu.VMEM((CH2, H), jnp.float32),
        pltpu.VMEM_SHARED((NV_PAD, H), jnp.float32),
        pltpu.SemaphoreType.DMA,
        pltpu.SemaphoreType.DMA,
    ],
  )


def _dot(a, b):
    return jnp.dot(a, b, preferred_element_type=jnp.float32)


def _tc1_body(a_ref, ce_ref, e_ref, w1_ref, b1_ref, w2_ref, b2_ref, o_ref):
    a = a_ref[0] + a_ref[1]
    cnt = ce_ref[0, :, 0:1] + ce_ref[1, :, 0:1]
    ind = (cnt > 0.0).astype(jnp.float32)
    am = a / jnp.maximum(cnt, 1.0)
    e = e_ref[...]
    me = _dot(am, w1_ref[0:H]) + ind * (_dot(e, w1_ref[H:2 * H]) + b1_ref[...])
    o_ref[...] = _dot(e, w2_ref[0:H]) + _dot(me, w2_ref[H:2 * H]) + b2_ref[...]


def _tc2_body(x_ref, g_ref, cv_ref, w3_ref, b3_ref, w4_ref, b4_ref, o_ref):
    g = g_ref[0] + g_ref[1]
    cnt = cv_ref[0, :, 0:1] + cv_ref[1, :, 0:1]
    row = lax.broadcasted_iota(jnp.int32, cnt.shape, 0)
    first = (pl.program_id(0) == 0) & (row == 0)
    cnt = cnt - jnp.where(first, PADTOT, 0.0)
    ind = (cnt > 0.0).astype(jnp.float32)
    gm = g / jnp.maximum(cnt, 1.0)
    x = x_ref[...]
    mv = ind * (_dot(x, w3_ref[0:H]) + b3_ref[...]) + _dot(gm, w3_ref[H:2 * H])
    o_ref[...] = _dot(x, w4_ref[0:H]) + _dot(mv, w4_ref[H:2 * H]) + b4_ref[...]


def _full(shape):
    return pl.BlockSpec(shape, lambda i: (0,) * len(shape))


def _rows(R):
    return pl.BlockSpec((R, H), lambda i: (i, 0))


def _make_tc1():
    R = 1000
    return pl.pallas_call(
        _tc1_body,
        grid=(NE // R,),
        in_specs=[
            pl.BlockSpec((NC, R, H), lambda i: (0, i, 0)),
            pl.BlockSpec((NC, R, 16), lambda i: (0, i, 0)),
            _rows(R),
            _full((2 * H, H)),
            _full((1, H)),
            _full((2 * H, H)),
            _full((1, H)),
        ],
        out_specs=_rows(R),
        out_shape=jax.ShapeDtypeStruct((NE, H), jnp.float32),
    )


def _make_tc2():
    R = 2000
    return pl.pallas_call(
        _tc2_body,
        grid=(NV // R,),
        in_specs=[
            _rows(R),
            pl.BlockSpec((NC, R, H), lambda i: (0, i, 0)),
            pl.BlockSpec((NC, R, 16), lambda i: (0, i, 0)),
            _full((2 * H, H)),
            _full((1, H)),
            _full((2 * H, H)),
            _full((1, H)),
        ],
        out_specs=_rows(R),
        out_shape=jax.ShapeDtypeStruct((NV, H), jnp.float32),
    )


@jax.jit
def _run(X, E, vertex, edges, W1, b1, W2, b2, W3, b3, W4, b4):
    vertex = vertex.astype(jnp.int32).reshape(NW, PAIRS_PER_W)
    edges = edges.astype(jnp.int32).reshape(NW, PAIRS_PER_W)
    v_g = jnp.pad(vertex, ((0, 0), (0, PADW - PAIRS_PER_W))
                  ).reshape(NW, NCHUNK1, CH1)
    e_s = jnp.pad(edges, ((0, 0), (0, PADW - PAIRS_PER_W)),
                  constant_values=NE_PAD - 1).reshape(NW, NCHUNK1, CH1)
    v_2 = vertex.reshape(NW, NCHUNK2, CH2)
    e_2 = edges.reshape(NW, NCHUNK2, CH2)

    a_p, ce_p, cv_p = _sc1()(X, v_g, e_s)
    a_p = a_p.reshape(NC, NE_PAD, H)
    ce_p = ce_p.reshape(NC, NE_PAD, 16)
    cv_p = cv_p.reshape(NC, NV_PAD, 16)
    e2 = _make_tc1()(a_p, ce_p, E,
                     W1, b1.reshape(1, H), W2, b2.reshape(1, H))
    g_p = _sc2()(e2, v_2, e_2)
    g_p = g_p.reshape(NC, NV_PAD, H)
    x2 = _make_tc2()(X, g_p, cv_p,
                     W3, b3.reshape(1, H), W4, b4.reshape(1, H))
    return x2, e2


def kernel(X, E, vertex, edges, W1, b1, W2, b2, W3, b3, W4, b4):
    return _run(X, E, vertex, edges, W1, b1, W2, b2, W3, b3, W4, b4)
